# Initial kernel scaffold; baseline (speedup 1.0000x reference)
import jax
import jax.numpy as jnp
from jax.experimental import pallas as pl


def kernel(pts, W_head, bias_head, W_b0, bias_b0, W_b1, bias_b1, W_b2, bias_b2, W_fusion):
    raise NotImplementedError("write your pallas kernel here")


# speedup vs baseline: 270.6285x; 270.6285x over previous
import functools

import jax
import jax.numpy as jnp
from jax import lax
from jax.experimental import pallas as pl
from jax.experimental.pallas import tpu as pltpu
from jax.experimental.pallas import tpu_sc as plsc

B = 8
N = 1024
K = 20
KPAD = 24
BN = B * N
RADIUS2 = 0.1 * 0.1
SLOPE = 0.2
EPS = 1e-5
MK = float(BN * K)

NC = 2
NS = 16
NW = NC * NS
RPW = BN // NW


def _leaky(x):
    return jnp.where(x >= 0, x, SLOPE * x)


def _d2_body(p_ref, o_ref):
    p = p_ref[0]
    pp = p * p
    ones = jnp.ones((1, 8), jnp.float32)
    dn = (((1,), (1,)), ((), ()))
    x2r = lax.dot_general(ones, pp, dn, preferred_element_type=jnp.float32)
    x2c = lax.dot_general(pp, ones, dn, preferred_element_type=jnp.float32)
    g = lax.dot_general(p, p, dn, preferred_element_type=jnp.float32)
    o_ref[0] = x2c + x2r - 2.0 * g


_d2_call = pl.pallas_call(
    _d2_body,
    grid=(B,),
    in_specs=[pl.BlockSpec((1, N, 8), lambda i: (i, 0, 0))],
    out_specs=pl.BlockSpec((1, N, N), lambda i: (i, 0, 0)),
    out_shape=jax.ShapeDtypeStruct((B, N, N), jnp.float32),
)


_G = 8


def _bq_body(d2_hbm, idx_hbm, dbuf, rowbuf, outbuf):
    cid = lax.axis_index("c")
    sid = lax.axis_index("s")
    wid = sid * NC + cid
    base = wid * RPW

    def group(gi, _):
        gbase = base + gi * _G
        pltpu.sync_copy(d2_hbm.at[pl.ds(gbase, _G)], dbuf)

        def row(gr, _):
            r = gbase + gr
            bglob = (r // N) * N

            r2v = jnp.full((16,), RADIUS2, jnp.float32)
            iv = lax.iota(jnp.int32, 16)
            zv = jnp.zeros((16,), jnp.int32)
            ov = jnp.full((16,), 1, jnp.int32)

            def chunk(c, cnt):
                vd = dbuf[gr, pl.ds(c * 16, 16)]
                m = vd <= r2v
                m32 = jnp.where(m, ov, zv)
                vidx = iv + jnp.full((16,), c * 16 + bglob, jnp.int32)
                pos = jnp.full((16,), cnt - 1, jnp.int32) + plsc.cumsum(m32)
                plsc.store_scatter(rowbuf, [pos], vidx, mask=m)
                return cnt + jnp.sum(m32)

            cnt = lax.fori_loop(0, N // 16, chunk, 0)
            cntv = jnp.full((16,), cnt, jnp.int32)
            ii0 = iv
            ii1 = iv + jnp.full((16,), 16, jnp.int32)
            sel0 = jnp.where(ii0 < cntv, ii0, zv)
            sel1 = jnp.where(ii1 < cntv, ii1, zv)
            outbuf[pl.ds(0, 16)] = plsc.load_gather(rowbuf, [sel0])
            outbuf[pl.ds(16, 16)] = plsc.load_gather(rowbuf, [sel1])
            pltpu.sync_copy(outbuf.at[pl.ds(0, KPAD)], idx_hbm.at[r])
            return 0

        lax.fori_loop(0, _G, row, 0)
        return 0

    lax.fori_loop(0, RPW // _G, group, 0)


_bq_call = functools.partial(
    pl.kernel,
    mesh=plsc.VectorSubcoreMesh(core_axis_name="c", subcore_axis_name="s"),
    compiler_params=pltpu.CompilerParams(needs_layout_passes=False, use_tc_tiling_on_sc=False),
    out_type=jax.ShapeDtypeStruct((BN, KPAD), jnp.int32),
    scratch_types=[
        pltpu.VMEM((_G, N), jnp.float32),
        pltpu.VMEM((N + 32,), jnp.int32),
        pltpu.VMEM((32,), jnp.int32),
    ],
)(_bq_body)


_BNB = 1024


def _prep_head_body(x_ref, w1t_ref, w2t_ref, b_ref, u_ref, v_ref):
    x = x_ref[...]
    a = w1t_ref[...] - w2t_ref[...]
    u_ref[...] = jnp.dot(x, a, preferred_element_type=jnp.float32) + b_ref[...]
    v_ref[...] = jnp.dot(x, w2t_ref[...], preferred_element_type=jnp.float32)


def _prep_mid_body(f_ref, s_ref, w1t_ref, w2t_ref, b_ref, u_ref, v_ref):
    st = s_ref[...]
    mean = jnp.sum(st[:NW], axis=0, keepdims=True) / MK
    sq = jnp.sum(st[NW:], axis=0, keepdims=True) / MK
    var = sq - mean * mean
    rstd = lax.rsqrt(var + EPS)
    x = (f_ref[...] - mean) * rstd
    a = w1t_ref[...] - w2t_ref[...]
    u_ref[...] = jnp.dot(x, a, preferred_element_type=jnp.float32) + b_ref[...]
    v_ref[...] = jnp.dot(x, w2t_ref[...], preferred_element_type=jnp.float32)


def _make_prep_head(cin, oc):
    return pl.pallas_call(
        _prep_head_body,
        grid=(BN // _BNB,),
        in_specs=[
            pl.BlockSpec((_BNB, cin), lambda i: (i, 0)),
            pl.BlockSpec((cin, oc), lambda i: (0, 0)),
            pl.BlockSpec((cin, oc), lambda i: (0, 0)),
            pl.BlockSpec((1, oc), lambda i: (0, 0)),
        ],
        out_specs=[
            pl.BlockSpec((_BNB, oc), lambda i: (i, 0)),
            pl.BlockSpec((_BNB, oc), lambda i: (i, 0)),
        ],
        out_shape=[
            jax.ShapeDtypeStruct((BN, oc), jnp.float32),
            jax.ShapeDtypeStruct((BN, oc), jnp.float32),
        ],
    )


def _make_prep_mid(cin, oc):
    return pl.pallas_call(
        _prep_mid_body,
        grid=(BN // _BNB,),
        in_specs=[
            pl.BlockSpec((_BNB, cin), lambda i: (i, 0)),
            pl.BlockSpec((2 * NW, cin), lambda i: (0, 0)),
            pl.BlockSpec((cin, oc), lambda i: (0, 0)),
            pl.BlockSpec((cin, oc), lambda i: (0, 0)),
            pl.BlockSpec((1, oc), lambda i: (0, 0)),
        ],
        out_specs=[
            pl.BlockSpec((_BNB, oc), lambda i: (i, 0)),
            pl.BlockSpec((_BNB, oc), lambda i: (i, 0)),
        ],
        out_shape=[
            jax.ShapeDtypeStruct((BN, oc), jnp.float32),
            jax.ShapeDtypeStruct((BN, oc), jnp.float32),
        ],
    )


def _make_gather(oc):
    nch = oc // 16

    def body(u_hbm, v_hbm, idx_hbm, fpre_hbm, stats_hbm,
             idxbuf, ubuf, vrows, accs, accq, outbuf, sem):
        cid = lax.axis_index("c")
        sid = lax.axis_index("s")
        wid = sid * NC + cid
        base = wid * RPW

        zero = jnp.zeros((16,), jnp.float32)
        for j in range(nch):
            accs[pl.ds(16 * j, 16)] = zero
            accq[pl.ds(16 * j, 16)] = zero

        def point(i, _):
            r = base + i
            pltpu.sync_copy(idx_hbm.at[r], idxbuf)
            pltpu.sync_copy(u_hbm.at[r], ubuf)
            pltpu.async_copy(v_hbm.at[idxbuf], vrows, sem).wait()
            for j in range(nch):
                sl = pl.ds(16 * j, 16)
                vu = ubuf[sl]
                vmax = jnp.full((16,), -jnp.inf, jnp.float32)
                vs = zero
                vq = zero
                for kk in range(K):
                    vv = vrows[kk, sl]
                    z = _leaky(vu + vv)
                    vs = vs + z
                    vq = vq + z * z
                    vmax = jnp.maximum(vmax, vv)
                outbuf[sl] = _leaky(vu + vmax)
                accs[sl] = accs[sl] + vs
                accq[sl] = accq[sl] + vq
            pltpu.sync_copy(outbuf, fpre_hbm.at[r])
            return 0

        lax.fori_loop(0, RPW, point, 0)
        pltpu.sync_copy(accs, stats_hbm.at[wid])
        pltpu.sync_copy(accq, stats_hbm.at[NW + wid])

    return functools.partial(
        pl.kernel,
        mesh=plsc.VectorSubcoreMesh(core_axis_name="c", subcore_axis_name="s"),
        compiler_params=pltpu.CompilerParams(needs_layout_passes=False, use_tc_tiling_on_sc=False),
        out_type=[
            jax.ShapeDtypeStruct((BN, oc), jnp.float32),
            jax.ShapeDtypeStruct((2 * NW, oc), jnp.float32),
        ],
        scratch_types=[
            pltpu.VMEM((KPAD,), jnp.int32),
            pltpu.VMEM((oc,), jnp.float32),
            pltpu.VMEM((KPAD, oc), jnp.float32),
            pltpu.VMEM((oc,), jnp.float32),
            pltpu.VMEM((oc,), jnp.float32),
            pltpu.VMEM((oc,), jnp.float32),
            pltpu.SemaphoreType.DMA,
        ],
    )(body)


_OCS = [64, 64, 128, 256]
_OB = 256


def _fusion_body(f1, s1, f2, s2, f3, s3, f4, s4, wt_ref, o_ref):
    wt = wt_ref[...]
    y = jnp.zeros((BN, _OB), jnp.float32)
    off = 0
    for f_ref, s_ref, c in ((f1, s1, _OCS[0]), (f2, s2, _OCS[1]),
                            (f3, s3, _OCS[2]), (f4, s4, _OCS[3])):
        st = s_ref[...]
        mean = jnp.sum(st[:NW], axis=0, keepdims=True) / MK
        sq = jnp.sum(st[NW:], axis=0, keepdims=True) / MK
        rstd = lax.rsqrt(sq - mean * mean + EPS)
        xn = (f_ref[...] - mean) * rstd
        y = y + jnp.dot(xn, wt[off:off + c], preferred_element_type=jnp.float32)
        off += c
    z = _leaky(y)
    m = jnp.sum(z, axis=0, keepdims=True) / BN
    q = jnp.sum(z * z, axis=0, keepdims=True) / BN
    rstd = lax.rsqrt(q - m * m + EPS)
    o_ref[...] = (z - m) * rstd


def _make_fusion():
    in_specs = []
    for c in _OCS:
        in_specs.append(pl.BlockSpec((BN, c), lambda i: (0, 0)))
        in_specs.append(pl.BlockSpec((2 * NW, c), lambda i: (0, 0)))
    in_specs.append(pl.BlockSpec((512, _OB), lambda i: (0, i)))
    return pl.pallas_call(
        _fusion_body,
        grid=(1024 // _OB,),
        in_specs=in_specs,
        out_specs=pl.BlockSpec((BN, _OB), lambda i: (0, i)),
        out_shape=jax.ShapeDtypeStruct((BN, 1024), jnp.float32),
    )


_prep_head = _make_prep_head(8, 64)
_prep_b0 = _make_prep_mid(64, 64)
_prep_b1 = _make_prep_mid(64, 128)
_prep_b2 = _make_prep_mid(128, 256)
_gather_64 = _make_gather(64)
_gather_128 = _make_gather(128)
_gather_256 = _make_gather(256)
_fusion = _make_fusion()


def _split_w(w, cin, pad_to=None):
    w1t = jnp.transpose(w[:, :cin])
    w2t = jnp.transpose(w[:, cin:])
    if pad_to is not None:
        padrows = ((0, pad_to - cin), (0, 0))
        w1t = jnp.pad(w1t, padrows)
        w2t = jnp.pad(w2t, padrows)
    return w1t, w2t


def kernel(pts, W_head, bias_head, W_b0, bias_b0, W_b1, bias_b1, W_b2,
           bias_b2, W_fusion):
    ptsf = pts.reshape(BN, 3)
    ptsp = jnp.pad(ptsf, ((0, 0), (0, 5)))
    d2 = _d2_call(ptsp.reshape(B, N, 8))
    idx = _bq_call(d2.reshape(BN, N))

    w1t, w2t = _split_w(W_head, 3, pad_to=8)
    u, v = _prep_head(ptsp, w1t, w2t, bias_head.reshape(1, -1))
    f1, s1 = _gather_64(u, v, idx)

    w1t, w2t = _split_w(W_b0, 64)
    u, v = _prep_b0(f1, s1, w1t, w2t, bias_b0.reshape(1, -1))
    f2, s2 = _gather_64(u, v, idx)

    w1t, w2t = _split_w(W_b1, 64)
    u, v = _prep_b1(f2, s2, w1t, w2t, bias_b1.reshape(1, -1))
    f3, s3 = _gather_128(u, v, idx)

    w1t, w2t = _split_w(W_b2, 128)
    u, v = _prep_b2(f3, s3, w1t, w2t, bias_b2.reshape(1, -1))
    f4, s4 = _gather_256(u, v, idx)

    out = _fusion(f1, s1, f2, s2, f3, s3, f4, s4, jnp.transpose(W_fusion))
    return out.reshape(B, N, 1024).transpose(0, 2, 1)


# speedup vs baseline: 831.0253x; 3.0707x over previous
import functools

import jax
import jax.numpy as jnp
from jax import lax
from jax.experimental import pallas as pl
from jax.experimental.pallas import tpu as pltpu
from jax.experimental.pallas import tpu_sc as plsc

B = 8
N = 1024
K = 20
KPAD = 24
BN = B * N
RADIUS2 = 0.1 * 0.1
SLOPE = 0.2
EPS = 1e-5
MK = float(BN * K)

NC = 2
NS = 16
NW = NC * NS
RPW = BN // NW


def _leaky(x):
    return jnp.where(x >= 0, x, SLOPE * x)


def _d2_body(p_ref, o_ref):
    p = p_ref[0]
    pp = p * p
    ones = jnp.ones((1, 8), jnp.float32)
    dn = (((1,), (1,)), ((), ()))
    x2r = lax.dot_general(ones, pp, dn, preferred_element_type=jnp.float32)
    x2c = lax.dot_general(pp, ones, dn, preferred_element_type=jnp.float32)
    g = lax.dot_general(p, p, dn, preferred_element_type=jnp.float32)
    o_ref[0] = x2c + x2r - 2.0 * g


_d2_call = pl.pallas_call(
    _d2_body,
    grid=(B,),
    in_specs=[pl.BlockSpec((1, N, 8), lambda i: (i, 0, 0))],
    out_specs=pl.BlockSpec((1, N, N), lambda i: (i, 0, 0)),
    out_shape=jax.ShapeDtypeStruct((B, N, N), jnp.float32),
)


_G2 = 8


def _bq_body(d2_hbm, idx_hbm, dbufA, dbufB, rowbuf, outbuf, semA, semB):
    cid = lax.axis_index("c")
    sid = lax.axis_index("s")
    wid = sid * NC + cid
    base = wid * RPW
    ngb = RPW // _G2

    def d_copy(t, buf, sem):
        return pltpu.make_async_copy(d2_hbm.at[pl.ds(base + t * _G2, _G2)],
                                     buf, sem)

    r2v = jnp.full((16,), RADIUS2, jnp.float32)
    iv = lax.iota(jnp.int32, 16)
    zv = jnp.zeros((16,), jnp.int32)
    ov = jnp.full((16,), 1, jnp.int32)
    sixteen = jnp.full((16,), 16, jnp.int32)

    def proc(t, buf, sem):
        d_copy(t, buf, sem).wait()
        for gr in range(_G2):
            bglob = ((base + t * _G2 + gr) // N) * N

            def chunk(c, cntv):
                vd = buf[gr, pl.ds(c * 16, 16)]
                m = vd <= r2v
                m32 = jnp.where(m, ov, zv)
                vidx = iv + jnp.full((16,), c * 16 + bglob, jnp.int32)
                pos = (cntv - ov) + plsc.cumsum(m32)
                plsc.store_scatter(rowbuf, [pos], vidx, mask=m)
                return cntv + plsc.all_reduce_population_count(m)

            cntv = lax.fori_loop(0, N // 16, chunk, zv)
            sel0 = jnp.where(iv < cntv, iv, zv)
            ii1 = iv + sixteen
            sel1 = jnp.where(ii1 < cntv, ii1, zv)
            outbuf[pl.ds(gr * K, 16)] = plsc.load_gather(rowbuf, [sel0])
            outbuf[pl.ds(gr * K + 16, 16)] = plsc.load_gather(rowbuf, [sel1])
        pltpu.sync_copy(outbuf.at[pl.ds(0, _G2 * K)],
                        idx_hbm.at[pl.ds((base + t * _G2) * K, _G2 * K)])

        @pl.when(t + 2 < ngb)
        def _():
            d_copy(t + 2, buf, sem).start()

    d_copy(0, dbufA, semA).start()
    d_copy(1, dbufB, semB).start()

    def pair(tt, _):
        proc(2 * tt, dbufA, semA)
        proc(2 * tt + 1, dbufB, semB)
        return 0

    lax.fori_loop(0, ngb // 2, pair, 0)


_bq_call = functools.partial(
    pl.kernel,
    mesh=plsc.VectorSubcoreMesh(core_axis_name="c", subcore_axis_name="s"),
    compiler_params=pltpu.CompilerParams(needs_layout_passes=False, use_tc_tiling_on_sc=False),
    out_type=jax.ShapeDtypeStruct((BN * K,), jnp.int32),
    scratch_types=[
        pltpu.VMEM((_G2, N), jnp.float32),
        pltpu.VMEM((_G2, N), jnp.float32),
        pltpu.VMEM((N + 32,), jnp.int32),
        pltpu.VMEM((_G2 * K + 16,), jnp.int32),
        pltpu.SemaphoreType.DMA,
        pltpu.SemaphoreType.DMA,
    ],
)(_bq_body)


_BNB = 1024


def _prep_head_body(x_ref, w1t_ref, w2t_ref, b_ref, u_ref, v_ref):
    x = x_ref[...]
    a = w1t_ref[...] - w2t_ref[...]
    u_ref[...] = jnp.dot(x, a, preferred_element_type=jnp.float32) + b_ref[...]
    v_ref[...] = jnp.dot(x, w2t_ref[...], preferred_element_type=jnp.float32)


def _prep_mid_body(f_ref, s_ref, w1t_ref, w2t_ref, b_ref, u_ref, v_ref):
    st = s_ref[...]
    mean = jnp.sum(st[:NW], axis=0, keepdims=True) / MK
    sq = jnp.sum(st[NW:], axis=0, keepdims=True) / MK
    var = sq - mean * mean
    rstd = lax.rsqrt(var + EPS)
    x = (f_ref[...] - mean) * rstd
    a = w1t_ref[...] - w2t_ref[...]
    u_ref[...] = jnp.dot(x, a, preferred_element_type=jnp.float32) + b_ref[...]
    v_ref[...] = jnp.dot(x, w2t_ref[...], preferred_element_type=jnp.float32)


def _make_prep_head(cin, oc):
    return pl.pallas_call(
        _prep_head_body,
        grid=(BN // _BNB,),
        in_specs=[
            pl.BlockSpec((_BNB, cin), lambda i: (i, 0)),
            pl.BlockSpec((cin, oc), lambda i: (0, 0)),
            pl.BlockSpec((cin, oc), lambda i: (0, 0)),
            pl.BlockSpec((1, oc), lambda i: (0, 0)),
        ],
        out_specs=[
            pl.BlockSpec((_BNB, oc), lambda i: (i, 0)),
            pl.BlockSpec((_BNB, oc), lambda i: (i, 0)),
        ],
        out_shape=[
            jax.ShapeDtypeStruct((BN, oc), jnp.float32),
            jax.ShapeDtypeStruct((BN, oc), jnp.float32),
        ],
    )


def _make_prep_mid(cin, oc):
    return pl.pallas_call(
        _prep_mid_body,
        grid=(BN // _BNB,),
        in_specs=[
            pl.BlockSpec((_BNB, cin), lambda i: (i, 0)),
            pl.BlockSpec((2 * NW, cin), lambda i: (0, 0)),
            pl.BlockSpec((cin, oc), lambda i: (0, 0)),
            pl.BlockSpec((cin, oc), lambda i: (0, 0)),
            pl.BlockSpec((1, oc), lambda i: (0, 0)),
        ],
        out_specs=[
            pl.BlockSpec((_BNB, oc), lambda i: (i, 0)),
            pl.BlockSpec((_BNB, oc), lambda i: (i, 0)),
        ],
        out_shape=[
            jax.ShapeDtypeStruct((BN, oc), jnp.float32),
            jax.ShapeDtypeStruct((BN, oc), jnp.float32),
        ],
    )


_GD = 4
_KI = _GD * K


def _make_gather(oc):
    nch = oc // 16
    ng = RPW // _GD

    def body(u_hbm, v_hbm, idx_hbm, fpre_hbm, stats_hbm,
             idxA, idxB, uA, uB, vrA, vrB, outA, outB, accs, accq,
             siA, siB, suA, suB, sgA, sgB, soA, soB):
        cid = lax.axis_index("c")
        sid = lax.axis_index("s")
        wid = sid * NC + cid
        pbase = wid * RPW

        zero = jnp.zeros((16,), jnp.float32)
        for j in range(nch):
            accs[pl.ds(16 * j, 16)] = zero
            accq[pl.ds(16 * j, 16)] = zero

        def idx_copy(g, buf, sem):
            return pltpu.make_async_copy(
                idx_hbm.at[pl.ds((pbase + g * _GD) * K, _KI)], buf, sem)

        def u_copy(g, buf, sem):
            return pltpu.make_async_copy(
                u_hbm.at[pl.ds(pbase + g * _GD, _GD)], buf, sem)

        def g_copy(idxbuf, buf, sem):
            return pltpu.make_async_copy(v_hbm.at[idxbuf], buf, sem)

        def o_copy(g, buf, sem):
            return pltpu.make_async_copy(
                buf, fpre_hbm.at[pl.ds(pbase + g * _GD, _GD)], sem)

        def compute(ubuf, vrbuf, outbuf):
            for gg in range(_GD):
                def jloop(j, _):
                    sl = pl.ds(16 * j, 16)
                    vu = ubuf[gg, sl]
                    vmax = vrbuf[gg * K, sl]
                    z0 = vu + vmax
                    z = jnp.maximum(z0, SLOPE * z0)
                    vs = z
                    vq = z * z
                    for kk in range(1, K):
                        vv = vrbuf[gg * K + kk, sl]
                        z0 = vu + vv
                        z = jnp.maximum(z0, SLOPE * z0)
                        vs = vs + z
                        vq = vq + z * z
                        vmax = jnp.maximum(vmax, vv)
                    f0 = vu + vmax
                    outbuf[gg, sl] = jnp.maximum(f0, SLOPE * f0)
                    accs[sl] = accs[sl] + vs
                    accq[sl] = accq[sl] + vq
                    return 0

                lax.fori_loop(0, nch, jloop, 0)

        def half(g, idxS, uS, vrS, outS, siS, suS, sgS, soS,
                 idxO, vrO, siO, sgO):
            g_copy(idxS, vrS, sgS).wait()

            @pl.when(g + 1 < ng)
            def _():
                idx_copy(g + 1, idxO, siO).wait()
                g_copy(idxO, vrO, sgO).start()

            u_copy(g, uS, suS).wait()

            @pl.when(g >= 2)
            def _():
                o_copy(g - 2, outS, soS).wait()

            compute(uS, vrS, outS)
            o_copy(g, outS, soS).start()

            @pl.when(g + 2 < ng)
            def _():
                idx_copy(g + 2, idxS, siS).start()
                u_copy(g + 2, uS, suS).start()

        idx_copy(0, idxA, siA).start()
        u_copy(0, uA, suA).start()
        idx_copy(1, idxB, siB).start()
        u_copy(1, uB, suB).start()
        idx_copy(0, idxA, siA).wait()
        g_copy(idxA, vrA, sgA).start()

        def pair(tt, _):
            g = 2 * tt
            half(g, idxA, uA, vrA, outA, siA, suA, sgA, soA,
                 idxB, vrB, siB, sgB)
            half(g + 1, idxB, uB, vrB, outB, siB, suB, sgB, soB,
                 idxA, vrA, siA, sgA)
            return 0

        lax.fori_loop(0, ng // 2, pair, 0)
        o_copy(ng - 2, outA, soA).wait()
        o_copy(ng - 1, outB, soB).wait()
        pltpu.sync_copy(accs, stats_hbm.at[wid])
        pltpu.sync_copy(accq, stats_hbm.at[NW + wid])

    return functools.partial(
        pl.kernel,
        mesh=plsc.VectorSubcoreMesh(core_axis_name="c", subcore_axis_name="s"),
        compiler_params=pltpu.CompilerParams(needs_layout_passes=False, use_tc_tiling_on_sc=False),
        out_type=[
            jax.ShapeDtypeStruct((BN, oc), jnp.float32),
            jax.ShapeDtypeStruct((2 * NW, oc), jnp.float32),
        ],
        scratch_types=[
            pltpu.VMEM((_KI,), jnp.int32),
            pltpu.VMEM((_KI,), jnp.int32),
            pltpu.VMEM((_GD, oc), jnp.float32),
            pltpu.VMEM((_GD, oc), jnp.float32),
            pltpu.VMEM((_KI, oc), jnp.float32),
            pltpu.VMEM((_KI, oc), jnp.float32),
            pltpu.VMEM((_GD, oc), jnp.float32),
            pltpu.VMEM((_GD, oc), jnp.float32),
            pltpu.VMEM((oc,), jnp.float32),
            pltpu.VMEM((oc,), jnp.float32),
            pltpu.SemaphoreType.DMA,
            pltpu.SemaphoreType.DMA,
            pltpu.SemaphoreType.DMA,
            pltpu.SemaphoreType.DMA,
            pltpu.SemaphoreType.DMA,
            pltpu.SemaphoreType.DMA,
            pltpu.SemaphoreType.DMA,
            pltpu.SemaphoreType.DMA,
        ],
    )(body)


_OCS = [64, 64, 128, 256]
_OB = 256


def _fusion_body(f1, s1, f2, s2, f3, s3, f4, s4, wt_ref, o_ref):
    wt = wt_ref[...]
    y = jnp.zeros((BN, _OB), jnp.float32)
    off = 0
    for f_ref, s_ref, c in ((f1, s1, _OCS[0]), (f2, s2, _OCS[1]),
                            (f3, s3, _OCS[2]), (f4, s4, _OCS[3])):
        st = s_ref[...]
        mean = jnp.sum(st[:NW], axis=0, keepdims=True) / MK
        sq = jnp.sum(st[NW:], axis=0, keepdims=True) / MK
        rstd = lax.rsqrt(sq - mean * mean + EPS)
        xn = (f_ref[...] - mean) * rstd
        y = y + jnp.dot(xn, wt[off:off + c], preferred_element_type=jnp.float32)
        off += c
    z = _leaky(y)
    m = jnp.sum(z, axis=0, keepdims=True) / BN
    q = jnp.sum(z * z, axis=0, keepdims=True) / BN
    rstd = lax.rsqrt(q - m * m + EPS)
    o_ref[...] = (z - m) * rstd


def _make_fusion():
    in_specs = []
    for c in _OCS:
        in_specs.append(pl.BlockSpec((BN, c), lambda i: (0, 0)))
        in_specs.append(pl.BlockSpec((2 * NW, c), lambda i: (0, 0)))
    in_specs.append(pl.BlockSpec((512, _OB), lambda i: (0, i)))
    return pl.pallas_call(
        _fusion_body,
        grid=(1024 // _OB,),
        in_specs=in_specs,
        out_specs=pl.BlockSpec((BN, _OB), lambda i: (0, i)),
        out_shape=jax.ShapeDtypeStruct((BN, 1024), jnp.float32),
    )


_prep_head = _make_prep_head(8, 64)
_prep_b0 = _make_prep_mid(64, 64)
_prep_b1 = _make_prep_mid(64, 128)
_prep_b2 = _make_prep_mid(128, 256)
_gather_64 = _make_gather(64)
_gather_128 = _make_gather(128)
_gather_256 = _make_gather(256)
_fusion = _make_fusion()


def _split_w(w, cin, pad_to=None):
    w1t = jnp.transpose(w[:, :cin])
    w2t = jnp.transpose(w[:, cin:])
    if pad_to is not None:
        padrows = ((0, pad_to - cin), (0, 0))
        w1t = jnp.pad(w1t, padrows)
        w2t = jnp.pad(w2t, padrows)
    return w1t, w2t


def kernel(pts, W_head, bias_head, W_b0, bias_b0, W_b1, bias_b1, W_b2,
           bias_b2, W_fusion):
    ptsf = pts.reshape(BN, 3)
    ptsp = jnp.pad(ptsf, ((0, 0), (0, 5)))
    d2 = _d2_call(ptsp.reshape(B, N, 8))
    idx = _bq_call(d2.reshape(BN, N))

    w1t, w2t = _split_w(W_head, 3, pad_to=8)
    u, v = _prep_head(ptsp, w1t, w2t, bias_head.reshape(1, -1))
    f1, s1 = _gather_64(u, v, idx)

    w1t, w2t = _split_w(W_b0, 64)
    u, v = _prep_b0(f1, s1, w1t, w2t, bias_b0.reshape(1, -1))
    f2, s2 = _gather_64(u, v, idx)

    w1t, w2t = _split_w(W_b1, 64)
    u, v = _prep_b1(f2, s2, w1t, w2t, bias_b1.reshape(1, -1))
    f3, s3 = _gather_128(u, v, idx)

    w1t, w2t = _split_w(W_b2, 128)
    u, v = _prep_b2(f3, s3, w1t, w2t, bias_b2.reshape(1, -1))
    f4, s4 = _gather_256(u, v, idx)

    out = _fusion(f1, s1, f2, s2, f3, s3, f4, s4, jnp.transpose(W_fusion))
    return out.reshape(B, N, 1024).transpose(0, 2, 1)


# speedup vs baseline: 836.6205x; 1.0067x over previous
import functools

import jax
import jax.numpy as jnp
from jax import lax
from jax.experimental import pallas as pl
from jax.experimental.pallas import tpu as pltpu
from jax.experimental.pallas import tpu_sc as plsc

B = 8
N = 1024
K = 20
KPAD = 24
BN = B * N
RADIUS2 = 0.1 * 0.1
SLOPE = 0.2
EPS = 1e-5
MK = float(BN * K)

NC = 2
NS = 16
NW = NC * NS
RPW = BN // NW


def _leaky(x):
    return jnp.where(x >= 0, x, SLOPE * x)


def _d2_body(p_ref, o_ref):
    p = p_ref[0]
    pp = p * p
    ones = jnp.ones((1, 8), jnp.float32)
    dn = (((1,), (1,)), ((), ()))
    x2r = lax.dot_general(ones, pp, dn, preferred_element_type=jnp.float32)
    x2c = lax.dot_general(pp, ones, dn, preferred_element_type=jnp.float32)
    g = lax.dot_general(p, p, dn, preferred_element_type=jnp.float32)
    o_ref[0] = x2c + x2r - 2.0 * g


_d2_call = pl.pallas_call(
    _d2_body,
    grid=(B,),
    in_specs=[pl.BlockSpec((1, N, 8), lambda i: (i, 0, 0))],
    out_specs=pl.BlockSpec((1, N, N), lambda i: (i, 0, 0)),
    out_shape=jax.ShapeDtypeStruct((B, N, N), jnp.float32),
)


_G2 = 8


def _bq_body(d2_hbm, idx_hbm, dbufA, dbufB, rowbuf, outbuf, semA, semB):
    cid = lax.axis_index("c")
    sid = lax.axis_index("s")
    wid = sid * NC + cid
    base = wid * RPW
    ngb = RPW // _G2

    def d_copy(t, buf, sem):
        return pltpu.make_async_copy(d2_hbm.at[pl.ds(base + t * _G2, _G2)],
                                     buf, sem)

    r2v = jnp.full((16,), RADIUS2, jnp.float32)
    iv = lax.iota(jnp.int32, 16)
    zv = jnp.zeros((16,), jnp.int32)
    ov = jnp.full((16,), 1, jnp.int32)
    sixteen = jnp.full((16,), 16, jnp.int32)

    def proc(t, buf, sem):
        d_copy(t, buf, sem).wait()
        for gr in range(_G2):
            bglob = ((base + t * _G2 + gr) // N) * N

            def chunk(c, cntv):
                vd = buf[gr, pl.ds(c * 16, 16)]
                m = vd <= r2v
                m32 = jnp.where(m, ov, zv)
                vidx = iv + jnp.full((16,), c * 16 + bglob, jnp.int32)
                pos = (cntv - ov) + plsc.cumsum(m32)
                plsc.store_scatter(rowbuf, [pos], vidx, mask=m)
                return cntv + plsc.all_reduce_population_count(m)

            cntv = lax.fori_loop(0, N // 16, chunk, zv, unroll=4)
            sel0 = jnp.where(iv < cntv, iv, zv)
            ii1 = iv + sixteen
            sel1 = jnp.where(ii1 < cntv, ii1, zv)
            outbuf[pl.ds(gr * K, 16)] = plsc.load_gather(rowbuf, [sel0])
            outbuf[pl.ds(gr * K + 16, 16)] = plsc.load_gather(rowbuf, [sel1])
        pltpu.sync_copy(outbuf.at[pl.ds(0, _G2 * K)],
                        idx_hbm.at[pl.ds((base + t * _G2) * K, _G2 * K)])

        @pl.when(t + 2 < ngb)
        def _():
            d_copy(t + 2, buf, sem).start()

    d_copy(0, dbufA, semA).start()
    d_copy(1, dbufB, semB).start()

    def pair(tt, _):
        proc(2 * tt, dbufA, semA)
        proc(2 * tt + 1, dbufB, semB)
        return 0

    lax.fori_loop(0, ngb // 2, pair, 0)


_bq_call = functools.partial(
    pl.kernel,
    mesh=plsc.VectorSubcoreMesh(core_axis_name="c", subcore_axis_name="s"),
    compiler_params=pltpu.CompilerParams(needs_layout_passes=False, use_tc_tiling_on_sc=False),
    out_type=jax.ShapeDtypeStruct((BN * K,), jnp.int32),
    scratch_types=[
        pltpu.VMEM((_G2, N), jnp.float32),
        pltpu.VMEM((_G2, N), jnp.float32),
        pltpu.VMEM((N + 32,), jnp.int32),
        pltpu.VMEM((_G2 * K + 16,), jnp.int32),
        pltpu.SemaphoreType.DMA,
        pltpu.SemaphoreType.DMA,
    ],
)(_bq_body)


_BNB = 1024


def _prep_head_body(x_ref, w1t_ref, w2t_ref, b_ref, u_ref, v_ref):
    x = x_ref[...]
    a = w1t_ref[...] - w2t_ref[...]
    u_ref[...] = jnp.dot(x, a, preferred_element_type=jnp.float32) + b_ref[...]
    v_ref[...] = jnp.dot(x, w2t_ref[...], preferred_element_type=jnp.float32)


def _prep_mid_body(f_ref, s_ref, w1t_ref, w2t_ref, b_ref, u_ref, v_ref):
    st = s_ref[...]
    mean = jnp.sum(st[:NW], axis=0, keepdims=True) / MK
    sq = jnp.sum(st[NW:], axis=0, keepdims=True) / MK
    var = sq - mean * mean
    rstd = lax.rsqrt(var + EPS)
    x = (f_ref[...] - mean) * rstd
    a = w1t_ref[...] - w2t_ref[...]
    u_ref[...] = jnp.dot(x, a, preferred_element_type=jnp.float32) + b_ref[...]
    v_ref[...] = jnp.dot(x, w2t_ref[...], preferred_element_type=jnp.float32)


def _make_prep_head(cin, oc):
    return pl.pallas_call(
        _prep_head_body,
        grid=(BN // _BNB,),
        in_specs=[
            pl.BlockSpec((_BNB, cin), lambda i: (i, 0)),
            pl.BlockSpec((cin, oc), lambda i: (0, 0)),
            pl.BlockSpec((cin, oc), lambda i: (0, 0)),
            pl.BlockSpec((1, oc), lambda i: (0, 0)),
        ],
        out_specs=[
            pl.BlockSpec((_BNB, oc), lambda i: (i, 0)),
            pl.BlockSpec((_BNB, oc), lambda i: (i, 0)),
        ],
        out_shape=[
            jax.ShapeDtypeStruct((BN, oc), jnp.float32),
            jax.ShapeDtypeStruct((BN, oc), jnp.float32),
        ],
    )


def _make_prep_mid(cin, oc):
    return pl.pallas_call(
        _prep_mid_body,
        grid=(BN // _BNB,),
        in_specs=[
            pl.BlockSpec((_BNB, cin), lambda i: (i, 0)),
            pl.BlockSpec((2 * NW, cin), lambda i: (0, 0)),
            pl.BlockSpec((cin, oc), lambda i: (0, 0)),
            pl.BlockSpec((cin, oc), lambda i: (0, 0)),
            pl.BlockSpec((1, oc), lambda i: (0, 0)),
        ],
        out_specs=[
            pl.BlockSpec((_BNB, oc), lambda i: (i, 0)),
            pl.BlockSpec((_BNB, oc), lambda i: (i, 0)),
        ],
        out_shape=[
            jax.ShapeDtypeStruct((BN, oc), jnp.float32),
            jax.ShapeDtypeStruct((BN, oc), jnp.float32),
        ],
    )


_GD = 4
_KI = _GD * K


def _make_gather(oc):
    nch = oc // 16
    ng = RPW // _GD

    def body(u_hbm, v_hbm, idx_hbm, fpre_hbm, stats_hbm,
             idxA, idxB, uA, uB, vrA, vrB, outA, outB, accs, accq,
             siA, siB, suA, suB, sgA, sgB, soA, soB):
        cid = lax.axis_index("c")
        sid = lax.axis_index("s")
        wid = sid * NC + cid
        pbase = wid * RPW

        zero = jnp.zeros((16,), jnp.float32)
        for j in range(nch):
            accs[pl.ds(16 * j, 16)] = zero
            accq[pl.ds(16 * j, 16)] = zero

        def idx_copy(g, buf, sem):
            return pltpu.make_async_copy(
                idx_hbm.at[pl.ds((pbase + g * _GD) * K, _KI)], buf, sem)

        def u_copy(g, buf, sem):
            return pltpu.make_async_copy(
                u_hbm.at[pl.ds(pbase + g * _GD, _GD)], buf, sem)

        def g_copy(idxbuf, buf, sem):
            return pltpu.make_async_copy(v_hbm.at[idxbuf], buf, sem)

        def o_copy(g, buf, sem):
            return pltpu.make_async_copy(
                buf, fpre_hbm.at[pl.ds(pbase + g * _GD, _GD)], sem)

        def compute(ubuf, vrbuf, outbuf):
            for gg in range(_GD):
                def jloop(j, _):
                    sl = pl.ds(16 * j, 16)
                    vu = ubuf[gg, sl]
                    vmax = vrbuf[gg * K, sl]
                    z0 = vu + vmax
                    z = jnp.maximum(z0, SLOPE * z0)
                    vs = z
                    vq = z * z
                    for kk in range(1, K):
                        vv = vrbuf[gg * K + kk, sl]
                        z0 = vu + vv
                        z = jnp.maximum(z0, SLOPE * z0)
                        vs = vs + z
                        vq = vq + z * z
                        vmax = jnp.maximum(vmax, vv)
                    f0 = vu + vmax
                    outbuf[gg, sl] = jnp.maximum(f0, SLOPE * f0)
                    accs[sl] = accs[sl] + vs
                    accq[sl] = accq[sl] + vq
                    return 0

                lax.fori_loop(0, nch, jloop, 0, unroll=2)

        def half(g, idxS, uS, vrS, outS, siS, suS, sgS, soS,
                 idxO, vrO, siO, sgO):
            g_copy(idxS, vrS, sgS).wait()

            @pl.when(g + 1 < ng)
            def _():
                idx_copy(g + 1, idxO, siO).wait()
                g_copy(idxO, vrO, sgO).start()

            u_copy(g, uS, suS).wait()

            @pl.when(g >= 2)
            def _():
                o_copy(g - 2, outS, soS).wait()

            compute(uS, vrS, outS)
            o_copy(g, outS, soS).start()

            @pl.when(g + 2 < ng)
            def _():
                idx_copy(g + 2, idxS, siS).start()
                u_copy(g + 2, uS, suS).start()

        idx_copy(0, idxA, siA).start()
        u_copy(0, uA, suA).start()
        idx_copy(1, idxB, siB).start()
        u_copy(1, uB, suB).start()
        idx_copy(0, idxA, siA).wait()
        g_copy(idxA, vrA, sgA).start()

        def pair(tt, _):
            g = 2 * tt
            half(g, idxA, uA, vrA, outA, siA, suA, sgA, soA,
                 idxB, vrB, siB, sgB)
            half(g + 1, idxB, uB, vrB, outB, siB, suB, sgB, soB,
                 idxA, vrA, siA, sgA)
            return 0

        lax.fori_loop(0, ng // 2, pair, 0)
        o_copy(ng - 2, outA, soA).wait()
        o_copy(ng - 1, outB, soB).wait()
        pltpu.sync_copy(accs, stats_hbm.at[wid])
        pltpu.sync_copy(accq, stats_hbm.at[NW + wid])

    return functools.partial(
        pl.kernel,
        mesh=plsc.VectorSubcoreMesh(core_axis_name="c", subcore_axis_name="s"),
        compiler_params=pltpu.CompilerParams(needs_layout_passes=False, use_tc_tiling_on_sc=False),
        out_type=[
            jax.ShapeDtypeStruct((BN, oc), jnp.float32),
            jax.ShapeDtypeStruct((2 * NW, oc), jnp.float32),
        ],
        scratch_types=[
            pltpu.VMEM((_KI,), jnp.int32),
            pltpu.VMEM((_KI,), jnp.int32),
            pltpu.VMEM((_GD, oc), jnp.float32),
            pltpu.VMEM((_GD, oc), jnp.float32),
            pltpu.VMEM((_KI, oc), jnp.float32),
            pltpu.VMEM((_KI, oc), jnp.float32),
            pltpu.VMEM((_GD, oc), jnp.float32),
            pltpu.VMEM((_GD, oc), jnp.float32),
            pltpu.VMEM((oc,), jnp.float32),
            pltpu.VMEM((oc,), jnp.float32),
            pltpu.SemaphoreType.DMA,
            pltpu.SemaphoreType.DMA,
            pltpu.SemaphoreType.DMA,
            pltpu.SemaphoreType.DMA,
            pltpu.SemaphoreType.DMA,
            pltpu.SemaphoreType.DMA,
            pltpu.SemaphoreType.DMA,
            pltpu.SemaphoreType.DMA,
        ],
    )(body)


_OCS = [64, 64, 128, 256]
_OB = 256


def _fusion_body(f1, s1, f2, s2, f3, s3, f4, s4, wt_ref, o_ref):
    wt = wt_ref[...]
    y = jnp.zeros((BN, _OB), jnp.float32)
    off = 0
    for f_ref, s_ref, c in ((f1, s1, _OCS[0]), (f2, s2, _OCS[1]),
                            (f3, s3, _OCS[2]), (f4, s4, _OCS[3])):
        st = s_ref[...]
        mean = jnp.sum(st[:NW], axis=0, keepdims=True) / MK
        sq = jnp.sum(st[NW:], axis=0, keepdims=True) / MK
        rstd = lax.rsqrt(sq - mean * mean + EPS)
        xn = (f_ref[...] - mean) * rstd
        y = y + jnp.dot(xn, wt[off:off + c], preferred_element_type=jnp.float32)
        off += c
    z = _leaky(y)
    m = jnp.sum(z, axis=0, keepdims=True) / BN
    q = jnp.sum(z * z, axis=0, keepdims=True) / BN
    rstd = lax.rsqrt(q - m * m + EPS)
    o_ref[...] = (z - m) * rstd


def _make_fusion():
    in_specs = []
    for c in _OCS:
        in_specs.append(pl.BlockSpec((BN, c), lambda i: (0, 0)))
        in_specs.append(pl.BlockSpec((2 * NW, c), lambda i: (0, 0)))
    in_specs.append(pl.BlockSpec((512, _OB), lambda i: (0, i)))
    return pl.pallas_call(
        _fusion_body,
        grid=(1024 // _OB,),
        in_specs=in_specs,
        out_specs=pl.BlockSpec((BN, _OB), lambda i: (0, i)),
        out_shape=jax.ShapeDtypeStruct((BN, 1024), jnp.float32),
    )


_prep_head = _make_prep_head(8, 64)
_prep_b0 = _make_prep_mid(64, 64)
_prep_b1 = _make_prep_mid(64, 128)
_prep_b2 = _make_prep_mid(128, 256)
_gather_64 = _make_gather(64)
_gather_128 = _make_gather(128)
_gather_256 = _make_gather(256)
_fusion = _make_fusion()


def _split_w(w, cin, pad_to=None):
    w1t = jnp.transpose(w[:, :cin])
    w2t = jnp.transpose(w[:, cin:])
    if pad_to is not None:
        padrows = ((0, pad_to - cin), (0, 0))
        w1t = jnp.pad(w1t, padrows)
        w2t = jnp.pad(w2t, padrows)
    return w1t, w2t


def kernel(pts, W_head, bias_head, W_b0, bias_b0, W_b1, bias_b1, W_b2,
           bias_b2, W_fusion):
    ptsf = pts.reshape(BN, 3)
    ptsp = jnp.pad(ptsf, ((0, 0), (0, 5)))
    d2 = _d2_call(ptsp.reshape(B, N, 8))
    idx = _bq_call(d2.reshape(BN, N))

    w1t, w2t = _split_w(W_head, 3, pad_to=8)
    u, v = _prep_head(ptsp, w1t, w2t, bias_head.reshape(1, -1))
    f1, s1 = _gather_64(u, v, idx)

    w1t, w2t = _split_w(W_b0, 64)
    u, v = _prep_b0(f1, s1, w1t, w2t, bias_b0.reshape(1, -1))
    f2, s2 = _gather_64(u, v, idx)

    w1t, w2t = _split_w(W_b1, 64)
    u, v = _prep_b1(f2, s2, w1t, w2t, bias_b1.reshape(1, -1))
    f3, s3 = _gather_128(u, v, idx)

    w1t, w2t = _split_w(W_b2, 128)
    u, v = _prep_b2(f3, s3, w1t, w2t, bias_b2.reshape(1, -1))
    f4, s4 = _gather_256(u, v, idx)

    out = _fusion(f1, s1, f2, s2, f3, s3, f4, s4, jnp.transpose(W_fusion))
    return out.reshape(B, N, 1024).transpose(0, 2, 1)
